# Initial kernel scaffold; baseline (speedup 1.0000x reference)
import jax
import jax.numpy as jnp
from jax.experimental import pallas as pl


def kernel(features, nodes, to_neighs):
    raise NotImplementedError("write your pallas kernel here")


# speedup vs baseline: 1.5408x; 1.5408x over previous
import functools

import jax
import jax.numpy as jnp
from jax import lax
from jax.experimental import pallas as pl
from jax.experimental.pallas import tpu as pltpu
from jax.experimental.pallas import tpu_sc as plsc

NC = 2
NS = 16
NW = NC * NS
DEG = 32
D = 128
GROW = 128
NODES_PER_CHUNK = GROW // DEG
DCH = D // 16


def _agg_body(b_per_w, nchunk, features, idx_all, out, idx_v, buf0, buf1,
              acc_v, sem0, sem1):
    wid = lax.axis_index("s") * NC + lax.axis_index("c")
    pltpu.sync_copy(idx_all.at[wid], idx_v)
    pltpu.async_copy(features.at[idx_v.at[0]], buf0, sem0)
    pltpu.async_copy(features.at[idx_v.at[1]], buf1, sem1)

    def compute_chunk(c, buf):
        def node_body(n, carry):
            row0 = n * DEG
            for dc in range(DCH):
                a = buf[row0, pl.ds(dc * 16, 16)]
                for j in range(1, DEG):
                    a = a + buf[row0 + j, pl.ds(dc * 16, 16)]
                acc_v[c * NODES_PER_CHUNK + n, pl.ds(dc * 16, 16)] = a
            return carry
        lax.fori_loop(0, NODES_PER_CHUNK, node_body, 0)

    def pair_body(i, carry):
        c0 = 2 * i
        pltpu.make_async_copy(features.at[idx_v.at[c0]], buf0, sem0).wait()
        compute_chunk(c0, buf0)

        @pl.when(c0 + 2 < nchunk)
        def _():
            pltpu.async_copy(features.at[idx_v.at[c0 + 2]], buf0, sem0)

        c1 = c0 + 1
        pltpu.make_async_copy(features.at[idx_v.at[c1]], buf1, sem1).wait()
        compute_chunk(c1, buf1)

        @pl.when(c1 + 2 < nchunk)
        def _():
            pltpu.async_copy(features.at[idx_v.at[c1 + 2]], buf1, sem1)

        return carry

    lax.fori_loop(0, nchunk // 2, pair_body, 0)
    pltpu.sync_copy(acc_v, out.at[pl.ds(wid * b_per_w, b_per_w)])


def kernel(features, nodes, to_neighs):
    del nodes
    B = to_neighs.shape[0]
    tn = to_neighs.astype(jnp.int32)
    bp_unit = NW * 8
    BP = ((B + bp_unit - 1) // bp_unit) * bp_unit
    b_per_w = BP // NW
    nchunk = b_per_w * DEG // GROW
    if BP != B:
        tn = jnp.pad(tn, ((0, BP - B), (0, 0)))
    idx_all = tn.reshape(NW, nchunk, GROW)

    mesh = plsc.VectorSubcoreMesh(core_axis_name="c", subcore_axis_name="s")
    run = pl.kernel(
        functools.partial(_agg_body, b_per_w, nchunk),
        out_type=jax.ShapeDtypeStruct((BP, D), jnp.float32),
        mesh=mesh,
        scratch_types=[
            pltpu.VMEM((nchunk, GROW), jnp.int32),
            pltpu.VMEM((GROW, D), jnp.float32),
            pltpu.VMEM((GROW, D), jnp.float32),
            pltpu.VMEM((b_per_w, D), jnp.float32),
            pltpu.SemaphoreType.DMA,
            pltpu.SemaphoreType.DMA,
        ],
    )
    out = run(features, idx_all)
    return out[:B]
